# baseline (speedup 1.0000x reference)
import functools

import jax
import jax.numpy as jnp
from jax import lax
from jax.experimental import pallas as pl
from jax.experimental.pallas import tpu as pltpu
from jax.experimental.pallas import tpu_sc as plsc

NC = 2
NS = 16
NT = NC * NS
LANES = 16
CH = 128
SLAB = 160
DH = 64


def _hist_kernel_body(dst_hbm, out_hbm, dst_v, hist_v, *, nchunk, np_pad):
    cid = lax.axis_index("c")
    sid = lax.axis_index("s")
    wid = cid * NS + sid
    nh = nchunk // 2
    pltpu.sync_copy(dst_hbm.at[sid, pl.ds(cid * nh, nh)], dst_v)
    zero16 = jnp.zeros((LANES,), jnp.float32)

    def zbody(i, _):
        hist_v[pl.ds(i * LANES, LANES)] = zero16
        return 0

    lax.fori_loop(0, np_pad // LANES, zbody, 0)
    ones16 = jnp.ones((LANES,), jnp.float32)

    def hbody(j, _):
        for k in range(CH // LANES):
            idx = dst_v[j, pl.ds(k * LANES, LANES)]
            plsc.addupdate_scatter(hist_v, [idx], ones16)
        return 0

    lax.fori_loop(0, nh, hbody, 0)
    pltpu.sync_copy(hist_v, out_hbm.at[wid])


def _scatter_kernel_body(src_hbm, dst_hbm, xs_hbm, out_hbm,
                         src_v, dst_v, b0, b1, b2, b3, acc_sh, gsems, ssems,
                         *, nchunk, np_pad):
    cid = lax.axis_index("c")
    sid = lax.axis_index("s")
    rows_per_tile = np_pad // NS
    bufs = (b0, b1, b2, b3)

    zero16 = jnp.zeros((LANES,), jnp.float32)

    def zbody(r, _):
        for k in range(DH // LANES):
            b0[r, pl.ds(k * LANES, LANES)] = zero16
        return 0

    lax.fori_loop(0, CH, zbody, 0)
    for k in range(rows_per_tile // CH):
        pltpu.sync_copy(b0, acc_sh.at[pl.ds(sid * rows_per_tile + k * CH, CH)])
    plsc.subcore_barrier()

    def gwait(t):
        return pltpu.make_async_copy(xs_hbm.at[src_v.at[0]], bufs[t],
                                     gsems.at[t])

    def swait(t):
        return pltpu.make_async_copy(bufs[t], acc_sh.at[dst_v.at[0]],
                                     ssems.at[t])

    for s in range(nchunk // SLAB):
        pltpu.sync_copy(src_hbm.at[cid, sid, pl.ds(s * SLAB, SLAB)], src_v)
        pltpu.sync_copy(dst_hbm.at[sid, pl.ds(s * SLAB, SLAB)], dst_v)
        for t in range(4):
            pltpu.async_copy(xs_hbm.at[src_v.at[t]], bufs[t], gsems.at[t])

        def cbody(i, _):
            j = 4 * i
            for t in range(4):
                gwait(t).wait()
                pltpu.async_copy(bufs[t], acc_sh.at[dst_v.at[j + t]],
                                 ssems.at[t], add=True)
            for t in range(4):
                swait(t).wait()
                pltpu.async_copy(xs_hbm.at[src_v.at[j + 4 + t]], bufs[t],
                                 gsems.at[t])
            return 0

        lax.fori_loop(0, SLAB // 4 - 1, cbody, 0)
        for t in range(4):
            gwait(t).wait()
            pltpu.async_copy(bufs[t], acc_sh.at[dst_v.at[SLAB - 4 + t]],
                             ssems.at[t], add=True)
        for t in range(4):
            swait(t).wait()

    plsc.subcore_barrier()
    pltpu.sync_copy(acc_sh.at[pl.ds(sid * rows_per_tile, rows_per_tile)],
                    out_hbm.at[cid, pl.ds(sid * rows_per_tile, rows_per_tile)])


def _scale_body(hist_ref, x_ref, d_ref, xs_ref):
    ones_col = jnp.ones((NT, 1), jnp.float32)
    deg = lax.dot_general(hist_ref[...], ones_col,
                          dimension_numbers=(((0,), (0,)), ((), ())),
                          preferred_element_type=jnp.float32,
                          precision=lax.Precision.HIGHEST) + 1.0
    d = lax.rsqrt(deg)
    d_ref[...] = d
    xsc = x_ref[...] * d
    xs_ref[0] = xsc[:, :DH]
    xs_ref[1] = xsc[:, DH:]


def _dense_body(p_ref, xs_ref, x_ref, d_ref,
                cw_ref, cb_ref, w1_ref, b1_ref, w2_ref, b2_ref,
                w3_ref, b3_ref, y_ref):
    dn = (((1,), (0,)), ((), ()))
    acc = jnp.concatenate([p_ref[0] + xs_ref[0], p_ref[1] + xs_ref[1]],
                          axis=-1)
    g = lax.dot_general(acc, cw_ref[...], dn,
                        preferred_element_type=jnp.float32)
    g = g * d_ref[...] + cb_ref[...]
    h = jnp.maximum(g, 0.0) + x_ref[...]
    h1 = lax.dot_general(h, w1_ref[...], dn,
                         preferred_element_type=jnp.float32) + b1_ref[...]
    h1 = jnp.maximum(h1, 0.0)
    h2 = lax.dot_general(h1, w2_ref[...], dn,
                         preferred_element_type=jnp.float32) + b2_ref[...]
    h2 = jnp.maximum(h2, 0.0)
    y_ref[...] = lax.dot_general(h2, w3_ref[...], dn,
                                 preferred_element_type=jnp.float32) + b3_ref[...]


def kernel(x, edge_index, conv_W, conv_b, lin1_W, lin1_b, lin2_W, lin2_b,
           lin3_W, lin3_b):
    n, dfeat = x.shape
    e = edge_index.shape[1]
    m = lin1_W.shape[1]
    ept = e // NS
    nchunk = -(-ept // CH)
    nchunk = -(-nchunk // (2 * SLAB)) * (2 * SLAB)
    ept_pad = nchunk * CH
    np_pad = -(-n // (NS * CH)) * (NS * CH)
    if np_pad == n:
        np_pad += NS * CH
    assert ept * NS == e and dfeat == 2 * DH

    src = edge_index[0].reshape(NS, ept)
    dst = edge_index[1].reshape(NS, ept)
    pad_n = ept_pad - ept
    src_g = jnp.concatenate(
        [src, jnp.zeros((NS, pad_n), jnp.int32)], axis=1
    ).reshape(NS, nchunk, CH)
    src_p = jnp.stack([src_g, src_g + np_pad])
    dst_p = jnp.concatenate(
        [dst, jnp.full((NS, pad_n), n, jnp.int32)], axis=1
    ).reshape(NS, nchunk, CH)
    x_pad = jnp.concatenate(
        [x, jnp.zeros((np_pad - n, dfeat), jnp.float32)], axis=0)

    mesh = plsc.VectorSubcoreMesh(core_axis_name="c", subcore_axis_name="s")
    sc_params = pltpu.CompilerParams(needs_layout_passes=False)

    hist = pl.kernel(
        functools.partial(_hist_kernel_body, nchunk=nchunk, np_pad=np_pad),
        out_type=jax.ShapeDtypeStruct((NT, np_pad), jnp.float32),
        mesh=mesh,
        compiler_params=sc_params,
        scratch_types=[
            pltpu.VMEM((nchunk // 2, CH), jnp.int32),
            pltpu.VMEM((np_pad,), jnp.float32),
        ],
    )(dst_p)

    bn = 2048
    grid_b = np_pad // bn
    d_col, xs2 = pl.pallas_call(
        _scale_body,
        grid=(grid_b,),
        in_specs=[
            pl.BlockSpec((NT, bn), lambda i: (0, i)),
            pl.BlockSpec((bn, dfeat), lambda i: (i, 0)),
        ],
        out_specs=[
            pl.BlockSpec((bn, 1), lambda i: (i, 0)),
            pl.BlockSpec((2, bn, DH), lambda i: (0, i, 0)),
        ],
        out_shape=[
            jax.ShapeDtypeStruct((np_pad, 1), jnp.float32),
            jax.ShapeDtypeStruct((2, np_pad, DH), jnp.float32),
        ],
    )(hist, x_pad)
    xs_tab = xs2.reshape(2 * np_pad, DH)

    parts = pl.kernel(
        functools.partial(_scatter_kernel_body, nchunk=nchunk,
                          np_pad=np_pad),
        out_type=jax.ShapeDtypeStruct((NC, np_pad, DH), jnp.float32),
        mesh=mesh,
        compiler_params=pltpu.CompilerParams(needs_layout_passes=False,
                                             use_tc_tiling_on_sc=False),
        scratch_types=[
            pltpu.VMEM((SLAB, CH), jnp.int32),
            pltpu.VMEM((SLAB, CH), jnp.int32),
            pltpu.VMEM((CH, DH), jnp.float32),
            pltpu.VMEM((CH, DH), jnp.float32),
            pltpu.VMEM((CH, DH), jnp.float32),
            pltpu.VMEM((CH, DH), jnp.float32),
            pltpu.VMEM_SHARED((np_pad, DH), jnp.float32),
            pltpu.SemaphoreType.DMA((4,)),
            pltpu.SemaphoreType.DMA((4,)),
        ],
    )(src_p, dst_p, xs_tab)

    bd = 2000
    grid_d = n // bd
    y = pl.pallas_call(
        _dense_body,
        grid=(grid_d,),
        in_specs=[
            pl.BlockSpec((NC, bd, DH), lambda i: (0, i, 0)),
            pl.BlockSpec((2, bd, DH), lambda i: (0, i, 0)),
            pl.BlockSpec((bd, dfeat), lambda i: (i, 0)),
            pl.BlockSpec((bd, 1), lambda i: (i, 0)),
            pl.BlockSpec((dfeat, dfeat), lambda i: (0, 0)),
            pl.BlockSpec((1, dfeat), lambda i: (0, 0)),
            pl.BlockSpec((dfeat, m), lambda i: (0, 0)),
            pl.BlockSpec((1, m), lambda i: (0, 0)),
            pl.BlockSpec((m, m), lambda i: (0, 0)),
            pl.BlockSpec((1, m), lambda i: (0, 0)),
            pl.BlockSpec((m, 1), lambda i: (0, 0)),
            pl.BlockSpec((1, 1), lambda i: (0, 0)),
        ],
        out_specs=pl.BlockSpec((bd, 1), lambda i: (i, 0)),
        out_shape=jax.ShapeDtypeStruct((n, 1), jnp.float32),
    )(parts, xs2, x, d_col,
      conv_W, conv_b.reshape(1, dfeat), lin1_W, lin1_b.reshape(1, m),
      lin2_W, lin2_b.reshape(1, m), lin3_W, lin3_b.reshape(1, 1))
    return y


# speedup vs baseline: 18.7828x; 18.7828x over previous
import functools

import jax
import jax.numpy as jnp
from jax import lax
from jax.experimental import pallas as pl
from jax.experimental.pallas import tpu as pltpu
from jax.experimental.pallas import tpu_sc as plsc

NC = 2
NS = 16
NT = NC * NS
LANES = 16
CH = 128
SLAB = 40
DH = 64


def _hist_kernel_body(dst_hbm, out_hbm, dst_v, hist_v, *, nchunk, np_pad):
    cid = lax.axis_index("c")
    sid = lax.axis_index("s")
    wid = cid * NS + sid
    nh = nchunk // 2
    pltpu.sync_copy(dst_hbm.at[sid, pl.ds(cid * nh, nh)], dst_v)
    zero16 = jnp.zeros((LANES,), jnp.float32)

    def zbody(i, _):
        hist_v[pl.ds(i * LANES, LANES)] = zero16
        return 0

    lax.fori_loop(0, np_pad // LANES, zbody, 0)
    ones16 = jnp.ones((LANES,), jnp.float32)

    def hbody(j, _):
        for k in range(CH // LANES):
            idx = dst_v[j, pl.ds(k * LANES, LANES)]
            plsc.addupdate_scatter(hist_v, [idx], ones16)
        return 0

    lax.fori_loop(0, nh, hbody, 0)
    pltpu.sync_copy(hist_v, out_hbm.at[wid])


def _scatter_kernel_body(src_hbm, dst_hbm, xs_hbm, out_hbm,
                         src_v, dst_v, b0, b1, b2, b3, acc_sh, gsems, ssems,
                         *, nchunk, np_pad):
    cid = lax.axis_index("c")
    sid = lax.axis_index("s")
    rows_per_tile = np_pad // NS
    bufs = (b0, b1, b2, b3)

    zero16 = jnp.zeros((LANES,), jnp.float32)

    def zbody(r, _):
        for k in range(DH // LANES):
            b0[r, pl.ds(k * LANES, LANES)] = zero16
        return 0

    lax.fori_loop(0, CH, zbody, 0)
    for k in range(rows_per_tile // CH):
        pltpu.sync_copy(b0, acc_sh.at[pl.ds(sid * rows_per_tile + k * CH, CH)])
    plsc.subcore_barrier()

    def gwait(t):
        return pltpu.make_async_copy(xs_hbm.at[src_v.at[0]], bufs[t],
                                     gsems.at[t])

    def swait(t):
        return pltpu.make_async_copy(bufs[t], acc_sh.at[dst_v.at[0]],
                                     ssems.at[t])

    for s in range(nchunk // SLAB):
        pltpu.sync_copy(src_hbm.at[cid, sid, pl.ds(s * SLAB, SLAB)], src_v)
        pltpu.sync_copy(dst_hbm.at[sid, pl.ds(s * SLAB, SLAB)], dst_v)
        for t in range(4):
            pltpu.async_copy(xs_hbm.at[src_v.at[t]], bufs[t], gsems.at[t])

        def cbody(i, _):
            j = 4 * i
            for t in range(4):
                gwait(t).wait()
                pltpu.async_copy(bufs[t], acc_sh.at[dst_v.at[j + t]],
                                 ssems.at[t], add=True)
            for t in range(4):
                swait(t).wait()
                pltpu.async_copy(xs_hbm.at[src_v.at[j + 4 + t]], bufs[t],
                                 gsems.at[t])
            return 0

        lax.fori_loop(0, SLAB // 4 - 1, cbody, 0)
        for t in range(4):
            gwait(t).wait()
            pltpu.async_copy(bufs[t], acc_sh.at[dst_v.at[SLAB - 4 + t]],
                             ssems.at[t], add=True)
        for t in range(4):
            swait(t).wait()

    plsc.subcore_barrier()
    pltpu.sync_copy(acc_sh.at[pl.ds(sid * rows_per_tile, rows_per_tile)],
                    out_hbm.at[cid, pl.ds(sid * rows_per_tile, rows_per_tile)])


def _scale_body(hist_ref, x_ref, d_ref, xs_ref):
    ones_col = jnp.ones((NT, 1), jnp.float32)
    deg = lax.dot_general(hist_ref[...], ones_col,
                          dimension_numbers=(((0,), (0,)), ((), ())),
                          preferred_element_type=jnp.float32,
                          precision=lax.Precision.HIGHEST) + 1.0
    d = lax.rsqrt(deg)
    d_ref[...] = d
    xsc = x_ref[...] * d
    xs_ref[0] = xsc[:, :DH]
    xs_ref[1] = xsc[:, DH:]


def _dense_body(p_ref, xs_ref, x_ref, d_ref,
                cw_ref, cb_ref, w1_ref, b1_ref, w2_ref, b2_ref,
                w3_ref, b3_ref, y_ref):
    dn = (((1,), (0,)), ((), ()))
    acc = jnp.concatenate([p_ref[0] + xs_ref[0], p_ref[1] + xs_ref[1]],
                          axis=-1)
    g = lax.dot_general(acc, cw_ref[...], dn,
                        preferred_element_type=jnp.float32)
    g = g * d_ref[...] + cb_ref[...]
    h = jnp.maximum(g, 0.0) + x_ref[...]
    h1 = lax.dot_general(h, w1_ref[...], dn,
                         preferred_element_type=jnp.float32) + b1_ref[...]
    h1 = jnp.maximum(h1, 0.0)
    h2 = lax.dot_general(h1, w2_ref[...], dn,
                         preferred_element_type=jnp.float32) + b2_ref[...]
    h2 = jnp.maximum(h2, 0.0)
    y_ref[...] = lax.dot_general(h2, w3_ref[...], dn,
                                 preferred_element_type=jnp.float32) + b3_ref[...]


def kernel(x, edge_index, conv_W, conv_b, lin1_W, lin1_b, lin2_W, lin2_b,
           lin3_W, lin3_b):
    n, dfeat = x.shape
    e = edge_index.shape[1]
    m = lin1_W.shape[1]
    ept = e // NS
    nchunk = -(-ept // CH)
    nchunk = -(-nchunk // (2 * SLAB)) * (2 * SLAB)
    ept_pad = nchunk * CH
    np_pad = -(-n // (NS * CH)) * (NS * CH)
    if np_pad == n:
        np_pad += NS * CH
    assert ept * NS == e and dfeat == 2 * DH

    src = edge_index[0].reshape(NS, ept)
    dst = edge_index[1].reshape(NS, ept)
    pad_n = ept_pad - ept
    src_g = jnp.concatenate(
        [src, jnp.zeros((NS, pad_n), jnp.int32)], axis=1
    ).reshape(NS, nchunk, CH)
    src_p = jnp.stack([src_g, src_g + np_pad])
    dst_p = jnp.concatenate(
        [dst, jnp.full((NS, pad_n), n, jnp.int32)], axis=1
    ).reshape(NS, nchunk, CH)
    x_pad = jnp.concatenate(
        [x, jnp.zeros((np_pad - n, dfeat), jnp.float32)], axis=0)

    mesh = plsc.VectorSubcoreMesh(core_axis_name="c", subcore_axis_name="s")
    sc_params = pltpu.CompilerParams(needs_layout_passes=False)

    hist = pl.kernel(
        functools.partial(_hist_kernel_body, nchunk=nchunk, np_pad=np_pad),
        out_type=jax.ShapeDtypeStruct((NT, np_pad), jnp.float32),
        mesh=mesh,
        compiler_params=sc_params,
        scratch_types=[
            pltpu.VMEM((nchunk // 2, CH), jnp.int32),
            pltpu.VMEM((np_pad,), jnp.float32),
        ],
    )(dst_p)

    bn = 2048
    grid_b = np_pad // bn
    d_col, xs2 = pl.pallas_call(
        _scale_body,
        grid=(grid_b,),
        in_specs=[
            pl.BlockSpec((NT, bn), lambda i: (0, i)),
            pl.BlockSpec((bn, dfeat), lambda i: (i, 0)),
        ],
        out_specs=[
            pl.BlockSpec((bn, 1), lambda i: (i, 0)),
            pl.BlockSpec((2, bn, DH), lambda i: (0, i, 0)),
        ],
        out_shape=[
            jax.ShapeDtypeStruct((np_pad, 1), jnp.float32),
            jax.ShapeDtypeStruct((2, np_pad, DH), jnp.float32),
        ],
    )(hist, x_pad)
    xs_tab = xs2.reshape(2 * np_pad, DH)

    parts = pl.kernel(
        functools.partial(_scatter_kernel_body, nchunk=nchunk,
                          np_pad=np_pad),
        out_type=jax.ShapeDtypeStruct((NC, np_pad, DH), jnp.float32),
        mesh=mesh,
        compiler_params=pltpu.CompilerParams(needs_layout_passes=False,
                                             use_tc_tiling_on_sc=False),
        scratch_types=[
            pltpu.VMEM((SLAB, CH), jnp.int32),
            pltpu.VMEM((SLAB, CH), jnp.int32),
            pltpu.VMEM((CH, DH), jnp.float32),
            pltpu.VMEM((CH, DH), jnp.float32),
            pltpu.VMEM((CH, DH), jnp.float32),
            pltpu.VMEM((CH, DH), jnp.float32),
            pltpu.VMEM_SHARED((np_pad, DH), jnp.float32),
            pltpu.SemaphoreType.DMA((4,)),
            pltpu.SemaphoreType.DMA((4,)),
        ],
    )(src_p, dst_p, xs_tab)

    bd = 2000
    grid_d = n // bd
    y = pl.pallas_call(
        _dense_body,
        grid=(grid_d,),
        in_specs=[
            pl.BlockSpec((NC, bd, DH), lambda i: (0, i, 0)),
            pl.BlockSpec((2, bd, DH), lambda i: (0, i, 0)),
            pl.BlockSpec((bd, dfeat), lambda i: (i, 0)),
            pl.BlockSpec((bd, 1), lambda i: (i, 0)),
            pl.BlockSpec((dfeat, dfeat), lambda i: (0, 0)),
            pl.BlockSpec((1, dfeat), lambda i: (0, 0)),
            pl.BlockSpec((dfeat, m), lambda i: (0, 0)),
            pl.BlockSpec((1, m), lambda i: (0, 0)),
            pl.BlockSpec((m, m), lambda i: (0, 0)),
            pl.BlockSpec((1, m), lambda i: (0, 0)),
            pl.BlockSpec((m, 1), lambda i: (0, 0)),
            pl.BlockSpec((1, 1), lambda i: (0, 0)),
        ],
        out_specs=pl.BlockSpec((bd, 1), lambda i: (i, 0)),
        out_shape=jax.ShapeDtypeStruct((n, 1), jnp.float32),
    )(parts, xs2, x, d_col,
      conv_W, conv_b.reshape(1, dfeat), lin1_W, lin1_b.reshape(1, m),
      lin2_W, lin2_b.reshape(1, m), lin3_W, lin3_b.reshape(1, 1))
    return y


# speedup vs baseline: 19.0299x; 1.0132x over previous
import functools

import jax
import jax.numpy as jnp
from jax import lax
from jax.experimental import pallas as pl
from jax.experimental.pallas import tpu as pltpu
from jax.experimental.pallas import tpu_sc as plsc

NC = 2
NS = 16
NT = NC * NS
LANES = 16
CH = 128
SLAB = 80
DH = 64


def _hist_kernel_body(dst_hbm, out_hbm, dst_v, hist_v, *, nchunk, np_pad):
    cid = lax.axis_index("c")
    sid = lax.axis_index("s")
    wid = cid * NS + sid
    nh = nchunk // 2
    pltpu.sync_copy(dst_hbm.at[sid, pl.ds(cid * nh, nh)], dst_v)
    zero16 = jnp.zeros((LANES,), jnp.float32)

    def zbody(i, _):
        hist_v[pl.ds(i * LANES, LANES)] = zero16
        return 0

    lax.fori_loop(0, np_pad // LANES, zbody, 0)
    ones16 = jnp.ones((LANES,), jnp.float32)

    def hbody(j, _):
        for k in range(CH // LANES):
            idx = dst_v[j, pl.ds(k * LANES, LANES)]
            plsc.addupdate_scatter(hist_v, [idx], ones16)
        return 0

    lax.fori_loop(0, nh, hbody, 0)
    pltpu.sync_copy(hist_v, out_hbm.at[wid])


def _scatter_kernel_body(src_hbm, dst_hbm, xs_hbm, out_hbm,
                         src_v, dst_v, b0, b1, b2, b3, acc_sh, gsems, ssems,
                         *, nchunk, np_pad):
    cid = lax.axis_index("c")
    sid = lax.axis_index("s")
    rows_per_tile = np_pad // NS
    bufs = (b0, b1, b2, b3)

    zero16 = jnp.zeros((LANES,), jnp.float32)

    def zbody(r, _):
        for k in range(DH // LANES):
            b0[r, pl.ds(k * LANES, LANES)] = zero16
        return 0

    lax.fori_loop(0, CH, zbody, 0)
    for k in range(rows_per_tile // CH):
        pltpu.sync_copy(b0, acc_sh.at[pl.ds(sid * rows_per_tile + k * CH, CH)])
    plsc.subcore_barrier()

    def gwait(t):
        return pltpu.make_async_copy(xs_hbm.at[src_v.at[0]], bufs[t],
                                     gsems.at[t])

    def swait(t):
        return pltpu.make_async_copy(bufs[t], acc_sh.at[dst_v.at[0]],
                                     ssems.at[t])

    for s in range(nchunk // SLAB):
        pltpu.sync_copy(src_hbm.at[cid, sid, pl.ds(s * SLAB, SLAB)], src_v)
        pltpu.sync_copy(dst_hbm.at[sid, pl.ds(s * SLAB, SLAB)], dst_v)
        for t in range(4):
            pltpu.async_copy(xs_hbm.at[src_v.at[t]], bufs[t], gsems.at[t])

        def cbody(i, _):
            j = 4 * i
            for t in range(4):
                gwait(t).wait()
                pltpu.async_copy(bufs[t], acc_sh.at[dst_v.at[j + t]],
                                 ssems.at[t], add=True)
            for t in range(4):
                swait(t).wait()
                pltpu.async_copy(xs_hbm.at[src_v.at[j + 4 + t]], bufs[t],
                                 gsems.at[t])
            return 0

        lax.fori_loop(0, SLAB // 4 - 1, cbody, 0)
        for t in range(4):
            gwait(t).wait()
            pltpu.async_copy(bufs[t], acc_sh.at[dst_v.at[SLAB - 4 + t]],
                             ssems.at[t], add=True)
        for t in range(4):
            swait(t).wait()

    plsc.subcore_barrier()
    pltpu.sync_copy(acc_sh.at[pl.ds(sid * rows_per_tile, rows_per_tile)],
                    out_hbm.at[cid, pl.ds(sid * rows_per_tile, rows_per_tile)])


def _scale_body(hist_ref, x_ref, d_ref, xs_ref):
    ones_col = jnp.ones((NT, 1), jnp.float32)
    deg = lax.dot_general(hist_ref[...], ones_col,
                          dimension_numbers=(((0,), (0,)), ((), ())),
                          preferred_element_type=jnp.float32,
                          precision=lax.Precision.HIGHEST) + 1.0
    d = lax.rsqrt(deg)
    d_ref[...] = d
    xsc = x_ref[...] * d
    xs_ref[0] = xsc[:, :DH]
    xs_ref[1] = xsc[:, DH:]


def _dense_body(p_ref, xs_ref, x_ref, d_ref,
                cw_ref, cb_ref, w1_ref, b1_ref, w2_ref, b2_ref,
                w3_ref, b3_ref, y_ref):
    dn = (((1,), (0,)), ((), ()))
    acc = jnp.concatenate([p_ref[0] + xs_ref[0], p_ref[1] + xs_ref[1]],
                          axis=-1)
    g = lax.dot_general(acc, cw_ref[...], dn,
                        preferred_element_type=jnp.float32)
    g = g * d_ref[...] + cb_ref[...]
    h = jnp.maximum(g, 0.0) + x_ref[...]
    h1 = lax.dot_general(h, w1_ref[...], dn,
                         preferred_element_type=jnp.float32) + b1_ref[...]
    h1 = jnp.maximum(h1, 0.0)
    h2 = lax.dot_general(h1, w2_ref[...], dn,
                         preferred_element_type=jnp.float32) + b2_ref[...]
    h2 = jnp.maximum(h2, 0.0)
    y_ref[...] = lax.dot_general(h2, w3_ref[...], dn,
                                 preferred_element_type=jnp.float32) + b3_ref[...]


def kernel(x, edge_index, conv_W, conv_b, lin1_W, lin1_b, lin2_W, lin2_b,
           lin3_W, lin3_b):
    n, dfeat = x.shape
    e = edge_index.shape[1]
    m = lin1_W.shape[1]
    ept = e // NS
    nchunk = -(-ept // CH)
    nchunk = -(-nchunk // (2 * SLAB)) * (2 * SLAB)
    ept_pad = nchunk * CH
    np_pad = -(-n // (NS * CH)) * (NS * CH)
    if np_pad == n:
        np_pad += NS * CH
    assert ept * NS == e and dfeat == 2 * DH

    src = edge_index[0].reshape(NS, ept)
    dst = edge_index[1].reshape(NS, ept)
    pad_n = ept_pad - ept
    src_g = jnp.concatenate(
        [src, jnp.zeros((NS, pad_n), jnp.int32)], axis=1
    ).reshape(NS, nchunk, CH)
    src_p = jnp.stack([src_g, src_g + np_pad])
    dst_p = jnp.concatenate(
        [dst, jnp.full((NS, pad_n), n, jnp.int32)], axis=1
    ).reshape(NS, nchunk, CH)
    x_pad = jnp.concatenate(
        [x, jnp.zeros((np_pad - n, dfeat), jnp.float32)], axis=0)

    mesh = plsc.VectorSubcoreMesh(core_axis_name="c", subcore_axis_name="s")
    sc_params = pltpu.CompilerParams(needs_layout_passes=False)

    hist = pl.kernel(
        functools.partial(_hist_kernel_body, nchunk=nchunk, np_pad=np_pad),
        out_type=jax.ShapeDtypeStruct((NT, np_pad), jnp.float32),
        mesh=mesh,
        compiler_params=sc_params,
        scratch_types=[
            pltpu.VMEM((nchunk // 2, CH), jnp.int32),
            pltpu.VMEM((np_pad,), jnp.float32),
        ],
    )(dst_p)

    bn = 2048
    grid_b = np_pad // bn
    d_col, xs2 = pl.pallas_call(
        _scale_body,
        grid=(grid_b,),
        in_specs=[
            pl.BlockSpec((NT, bn), lambda i: (0, i)),
            pl.BlockSpec((bn, dfeat), lambda i: (i, 0)),
        ],
        out_specs=[
            pl.BlockSpec((bn, 1), lambda i: (i, 0)),
            pl.BlockSpec((2, bn, DH), lambda i: (0, i, 0)),
        ],
        out_shape=[
            jax.ShapeDtypeStruct((np_pad, 1), jnp.float32),
            jax.ShapeDtypeStruct((2, np_pad, DH), jnp.float32),
        ],
    )(hist, x_pad)
    xs_tab = xs2.reshape(2 * np_pad, DH)

    parts = pl.kernel(
        functools.partial(_scatter_kernel_body, nchunk=nchunk,
                          np_pad=np_pad),
        out_type=jax.ShapeDtypeStruct((NC, np_pad, DH), jnp.float32),
        mesh=mesh,
        compiler_params=pltpu.CompilerParams(needs_layout_passes=False,
                                             use_tc_tiling_on_sc=False),
        scratch_types=[
            pltpu.VMEM((SLAB, CH), jnp.int32),
            pltpu.VMEM((SLAB, CH), jnp.int32),
            pltpu.VMEM((CH, DH), jnp.float32),
            pltpu.VMEM((CH, DH), jnp.float32),
            pltpu.VMEM((CH, DH), jnp.float32),
            pltpu.VMEM((CH, DH), jnp.float32),
            pltpu.VMEM_SHARED((np_pad, DH), jnp.float32),
            pltpu.SemaphoreType.DMA((4,)),
            pltpu.SemaphoreType.DMA((4,)),
        ],
    )(src_p, dst_p, xs_tab)

    bd = 2000
    grid_d = n // bd
    y = pl.pallas_call(
        _dense_body,
        grid=(grid_d,),
        in_specs=[
            pl.BlockSpec((NC, bd, DH), lambda i: (0, i, 0)),
            pl.BlockSpec((2, bd, DH), lambda i: (0, i, 0)),
            pl.BlockSpec((bd, dfeat), lambda i: (i, 0)),
            pl.BlockSpec((bd, 1), lambda i: (i, 0)),
            pl.BlockSpec((dfeat, dfeat), lambda i: (0, 0)),
            pl.BlockSpec((1, dfeat), lambda i: (0, 0)),
            pl.BlockSpec((dfeat, m), lambda i: (0, 0)),
            pl.BlockSpec((1, m), lambda i: (0, 0)),
            pl.BlockSpec((m, m), lambda i: (0, 0)),
            pl.BlockSpec((1, m), lambda i: (0, 0)),
            pl.BlockSpec((m, 1), lambda i: (0, 0)),
            pl.BlockSpec((1, 1), lambda i: (0, 0)),
        ],
        out_specs=pl.BlockSpec((bd, 1), lambda i: (i, 0)),
        out_shape=jax.ShapeDtypeStruct((n, 1), jnp.float32),
    )(parts, xs2, x, d_col,
      conv_W, conv_b.reshape(1, dfeat), lin1_W, lin1_b.reshape(1, m),
      lin2_W, lin2_b.reshape(1, m), lin3_W, lin3_b.reshape(1, 1))
    return y


# speedup vs baseline: 19.3915x; 1.0190x over previous
import functools

import jax
import jax.numpy as jnp
from jax import lax
from jax.experimental import pallas as pl
from jax.experimental.pallas import tpu as pltpu
from jax.experimental.pallas import tpu_sc as plsc

NC = 2
NS = 16
NT = NC * NS
LANES = 16
CH = 128
SLAB = 80
DH = 64


def _hist_kernel_body(dst_hbm, out_hbm, dst_v, hist_v, *, nchunk, np_pad):
    cid = lax.axis_index("c")
    sid = lax.axis_index("s")
    wid = cid * NS + sid
    nh = nchunk // 2
    pltpu.sync_copy(dst_hbm.at[sid, pl.ds(cid * nh, nh)], dst_v)
    zero16 = jnp.zeros((LANES,), jnp.float32)

    def zbody(i, _):
        hist_v[pl.ds(i * LANES, LANES)] = zero16
        return 0

    lax.fori_loop(0, np_pad // LANES, zbody, 0)
    ones16 = jnp.ones((LANES,), jnp.float32)

    def hbody(j, _):
        for k in range(CH // LANES):
            idx = dst_v[j, pl.ds(k * LANES, LANES)]
            plsc.addupdate_scatter(hist_v, [idx], ones16)
        return 0

    lax.fori_loop(0, nh, hbody, 0)
    pltpu.sync_copy(hist_v, out_hbm.at[wid])


def _scatter_kernel_body(src_hbm, dst_hbm, xs_hbm, out_hbm,
                         src_v, dst_v, b0, b1, b2, b3, acc_sh, gsems, ssems,
                         *, nchunk, np_pad):
    cid = lax.axis_index("c")
    sid = lax.axis_index("s")
    rows_per_tile = np_pad // NS
    bufs = (b0, b1, b2, b3)

    zero16 = jnp.zeros((LANES,), jnp.float32)

    def zbody(r, _):
        for k in range(DH // LANES):
            b0[r, pl.ds(k * LANES, LANES)] = zero16
        return 0

    lax.fori_loop(0, CH, zbody, 0)
    for k in range(rows_per_tile // CH):
        pltpu.sync_copy(b0, acc_sh.at[pl.ds(sid * rows_per_tile + k * CH, CH)])
    plsc.subcore_barrier()

    def gwait(t):
        return pltpu.make_async_copy(xs_hbm.at[src_v.at[0]], bufs[t],
                                     gsems.at[t])

    def swait(t):
        return pltpu.make_async_copy(bufs[t], acc_sh.at[dst_v.at[0]],
                                     ssems.at[t])

    for s in range(nchunk // SLAB):
        pltpu.sync_copy(src_hbm.at[cid, sid, pl.ds(s * SLAB, SLAB)], src_v)
        pltpu.sync_copy(dst_hbm.at[sid, pl.ds(s * SLAB, SLAB)], dst_v)
        for t in range(4):
            pltpu.async_copy(xs_hbm.at[src_v.at[t]], bufs[t], gsems.at[t])

        def cbody(i, _):
            j = 4 * i
            for t in range(4):
                gwait(t).wait()
                pltpu.async_copy(bufs[t], acc_sh.at[dst_v.at[j + t]],
                                 ssems.at[t], add=True)
            for t in range(4):
                swait(t).wait()
                pltpu.async_copy(xs_hbm.at[src_v.at[j + 4 + t]], bufs[t],
                                 gsems.at[t])
            return 0

        lax.fori_loop(0, SLAB // 4 - 1, cbody, 0)
        for t in range(4):
            gwait(t).wait()
            pltpu.async_copy(bufs[t], acc_sh.at[dst_v.at[SLAB - 4 + t]],
                             ssems.at[t], add=True)
        for t in range(4):
            swait(t).wait()

    plsc.subcore_barrier()
    pltpu.sync_copy(acc_sh.at[pl.ds(sid * rows_per_tile, rows_per_tile)],
                    out_hbm.at[cid, pl.ds(sid * rows_per_tile, rows_per_tile)])


def _scale_body(hist_ref, x_ref, d_ref, xs_ref):
    ones_col = jnp.ones((NT, 1), jnp.float32)
    deg = lax.dot_general(hist_ref[...], ones_col,
                          dimension_numbers=(((0,), (0,)), ((), ())),
                          preferred_element_type=jnp.float32,
                          precision=lax.Precision.HIGHEST) + 1.0
    d = lax.rsqrt(deg)
    d_ref[...] = d
    xsc = x_ref[...] * d
    xs_ref[0] = xsc[:, :DH]
    xs_ref[1] = xsc[:, DH:]


def _dense_body(p_ref, xs_ref, x_ref, d_ref,
                cw_ref, cb_ref, w1_ref, b1_ref, w2_ref, b2_ref,
                w3_ref, b3_ref, y_ref):
    dn = (((1,), (0,)), ((), ()))
    acc = jnp.concatenate([p_ref[0] + xs_ref[0], p_ref[1] + xs_ref[1]],
                          axis=-1)
    g = lax.dot_general(acc, cw_ref[...], dn,
                        preferred_element_type=jnp.float32)
    g = g * d_ref[...] + cb_ref[...]
    h = jnp.maximum(g, 0.0) + x_ref[...]
    h1 = lax.dot_general(h, w1_ref[...], dn,
                         preferred_element_type=jnp.float32) + b1_ref[...]
    h1 = jnp.maximum(h1, 0.0)
    h2 = lax.dot_general(h1, w2_ref[...], dn,
                         preferred_element_type=jnp.float32) + b2_ref[...]
    h2 = jnp.maximum(h2, 0.0)
    y_ref[...] = lax.dot_general(h2, w3_ref[...], dn,
                                 preferred_element_type=jnp.float32) + b3_ref[...]


def kernel(x, edge_index, conv_W, conv_b, lin1_W, lin1_b, lin2_W, lin2_b,
           lin3_W, lin3_b):
    n, dfeat = x.shape
    e = edge_index.shape[1]
    m = lin1_W.shape[1]
    ept = e // NS
    nchunk = -(-ept // CH)
    nchunk = -(-nchunk // (2 * SLAB)) * (2 * SLAB)
    ept_pad = nchunk * CH
    np_pad = -(-n // (NS * CH)) * (NS * CH)
    if np_pad == n:
        np_pad += NS * CH
    assert ept * NS == e and dfeat == 2 * DH

    src = edge_index[0].reshape(NS, ept)
    dst = edge_index[1].reshape(NS, ept)
    pad_n = ept_pad - ept
    src_g = jnp.concatenate(
        [src, jnp.zeros((NS, pad_n), jnp.int32)], axis=1
    ).reshape(NS, nchunk, CH)
    src_p = jnp.stack([src_g, src_g + np_pad])
    dst_p = jnp.concatenate(
        [dst, jnp.full((NS, pad_n), n, jnp.int32)], axis=1
    ).reshape(NS, nchunk, CH)
    mesh = plsc.VectorSubcoreMesh(core_axis_name="c", subcore_axis_name="s")
    sc_params = pltpu.CompilerParams(needs_layout_passes=False)

    hist = pl.kernel(
        functools.partial(_hist_kernel_body, nchunk=nchunk, np_pad=np_pad),
        out_type=jax.ShapeDtypeStruct((NT, np_pad), jnp.float32),
        mesh=mesh,
        compiler_params=sc_params,
        scratch_types=[
            pltpu.VMEM((nchunk // 2, CH), jnp.int32),
            pltpu.VMEM((np_pad,), jnp.float32),
        ],
    )(dst_p)

    bn = 2048
    grid_b = np_pad // bn
    d_col, xs2 = pl.pallas_call(
        _scale_body,
        grid=(grid_b,),
        in_specs=[
            pl.BlockSpec((NT, bn), lambda i: (0, i)),
            pl.BlockSpec((bn, dfeat), lambda i: (i, 0)),
        ],
        out_specs=[
            pl.BlockSpec((bn, 1), lambda i: (i, 0)),
            pl.BlockSpec((2, bn, DH), lambda i: (0, i, 0)),
        ],
        out_shape=[
            jax.ShapeDtypeStruct((np_pad, 1), jnp.float32),
            jax.ShapeDtypeStruct((2, np_pad, DH), jnp.float32),
        ],
    )(hist, x)
    xs_tab = xs2.reshape(2 * np_pad, DH)

    parts = pl.kernel(
        functools.partial(_scatter_kernel_body, nchunk=nchunk,
                          np_pad=np_pad),
        out_type=jax.ShapeDtypeStruct((NC, np_pad, DH), jnp.float32),
        mesh=mesh,
        compiler_params=pltpu.CompilerParams(needs_layout_passes=False,
                                             use_tc_tiling_on_sc=False),
        scratch_types=[
            pltpu.VMEM((SLAB, CH), jnp.int32),
            pltpu.VMEM((SLAB, CH), jnp.int32),
            pltpu.VMEM((CH, DH), jnp.float32),
            pltpu.VMEM((CH, DH), jnp.float32),
            pltpu.VMEM((CH, DH), jnp.float32),
            pltpu.VMEM((CH, DH), jnp.float32),
            pltpu.VMEM_SHARED((np_pad, DH), jnp.float32),
            pltpu.SemaphoreType.DMA((4,)),
            pltpu.SemaphoreType.DMA((4,)),
        ],
    )(src_p, dst_p, xs_tab)

    bd = 2000
    grid_d = n // bd
    y = pl.pallas_call(
        _dense_body,
        grid=(grid_d,),
        in_specs=[
            pl.BlockSpec((NC, bd, DH), lambda i: (0, i, 0)),
            pl.BlockSpec((2, bd, DH), lambda i: (0, i, 0)),
            pl.BlockSpec((bd, dfeat), lambda i: (i, 0)),
            pl.BlockSpec((bd, 1), lambda i: (i, 0)),
            pl.BlockSpec((dfeat, dfeat), lambda i: (0, 0)),
            pl.BlockSpec((1, dfeat), lambda i: (0, 0)),
            pl.BlockSpec((dfeat, m), lambda i: (0, 0)),
            pl.BlockSpec((1, m), lambda i: (0, 0)),
            pl.BlockSpec((m, m), lambda i: (0, 0)),
            pl.BlockSpec((1, m), lambda i: (0, 0)),
            pl.BlockSpec((m, 1), lambda i: (0, 0)),
            pl.BlockSpec((1, 1), lambda i: (0, 0)),
        ],
        out_specs=pl.BlockSpec((bd, 1), lambda i: (i, 0)),
        out_shape=jax.ShapeDtypeStruct((n, 1), jnp.float32),
    )(parts, xs2, x, d_col,
      conv_W, conv_b.reshape(1, dfeat), lin1_W, lin1_b.reshape(1, m),
      lin2_W, lin2_b.reshape(1, m), lin3_W, lin3_b.reshape(1, 1))
    return y


# speedup vs baseline: 19.4776x; 1.0044x over previous
import functools

import jax
import jax.numpy as jnp
from jax import lax
from jax.experimental import pallas as pl
from jax.experimental.pallas import tpu as pltpu
from jax.experimental.pallas import tpu_sc as plsc

NC = 2
NS = 16
NT = NC * NS
LANES = 16
CH = 128
SLAB = 80
DH = 64


def _hist_kernel_body(dst_hbm, out_hbm, dst_v, hist_v, *, nchunk, np_pad):
    cid = lax.axis_index("c")
    sid = lax.axis_index("s")
    wid = cid * NS + sid
    nh = nchunk // 2
    pltpu.sync_copy(dst_hbm.at[sid, pl.ds(cid * nh, nh)], dst_v)
    zero16 = jnp.zeros((LANES,), jnp.float32)

    def zbody(i, _):
        hist_v[pl.ds(i * LANES, LANES)] = zero16
        return 0

    lax.fori_loop(0, np_pad // LANES, zbody, 0)
    ones16 = jnp.ones((LANES,), jnp.float32)

    def hbody(j, _):
        for k in range(CH // LANES):
            idx = dst_v[j, pl.ds(k * LANES, LANES)]
            plsc.addupdate_scatter(hist_v, [idx], ones16)
        return 0

    lax.fori_loop(0, nh, hbody, 0)
    pltpu.sync_copy(hist_v, out_hbm.at[wid])


def _scatter_kernel_body(src_hbm, dst_hbm, xs_hbm, out_hbm,
                         src_v, dst_v, b0, b1, b2, b3, acc_sh, gsems, ssems,
                         *, nchunk, np_pad):
    cid = lax.axis_index("c")
    sid = lax.axis_index("s")
    rows_per_tile = np_pad // NS
    bufs = (b0, b1, b2, b3)

    zero16 = jnp.zeros((LANES,), jnp.float32)

    def zbody(r, _):
        for k in range(DH // LANES):
            b0[r, pl.ds(k * LANES, LANES)] = zero16
        return 0

    lax.fori_loop(0, CH, zbody, 0)
    for k in range(rows_per_tile // CH):
        pltpu.sync_copy(b0, acc_sh.at[pl.ds(sid * rows_per_tile + k * CH, CH)])
    plsc.subcore_barrier()

    def gwait(t):
        return pltpu.make_async_copy(xs_hbm.at[src_v.at[0]], bufs[t],
                                     gsems.at[t])

    def swait(t):
        return pltpu.make_async_copy(bufs[t], acc_sh.at[dst_v.at[0]],
                                     ssems.at[t])

    for s in range(nchunk // SLAB):
        pltpu.sync_copy(src_hbm.at[cid, sid, pl.ds(s * SLAB, SLAB)], src_v)
        pltpu.sync_copy(dst_hbm.at[sid, pl.ds(s * SLAB, SLAB)], dst_v)
        for t in range(4):
            pltpu.async_copy(xs_hbm.at[src_v.at[t]], bufs[t], gsems.at[t])

        def cbody(i, _):
            j = 4 * i
            for t in range(4):
                gwait(t).wait()
                pltpu.async_copy(bufs[t], acc_sh.at[dst_v.at[j + t]],
                                 ssems.at[t], add=True)
            for t in range(4):
                swait(t).wait()
                pltpu.async_copy(xs_hbm.at[src_v.at[j + 4 + t]], bufs[t],
                                 gsems.at[t])
            return 0

        lax.fori_loop(0, SLAB // 4 - 1, cbody, 0)
        for t in range(4):
            gwait(t).wait()
            pltpu.async_copy(bufs[t], acc_sh.at[dst_v.at[SLAB - 4 + t]],
                             ssems.at[t], add=True)
        for t in range(4):
            swait(t).wait()

    plsc.subcore_barrier()
    pltpu.sync_copy(acc_sh.at[pl.ds(sid * rows_per_tile, rows_per_tile)],
                    out_hbm.at[cid, pl.ds(sid * rows_per_tile, rows_per_tile)])


def _scale_body(hist_ref, x_ref, d_ref, xs_ref):
    ones_col = jnp.ones((NT, 1), jnp.float32)
    deg = lax.dot_general(hist_ref[...], ones_col,
                          dimension_numbers=(((0,), (0,)), ((), ())),
                          preferred_element_type=jnp.float32,
                          precision=lax.Precision.HIGHEST) + 1.0
    d = lax.rsqrt(deg)
    d_ref[...] = d
    xsc = x_ref[...] * d
    xs_ref[0] = xsc[:, :DH]
    xs_ref[1] = xsc[:, DH:]


def _dense_body(p_ref, xs_ref, x_ref, d_ref,
                cw_ref, cb_ref, w1_ref, b1_ref, w2_ref, b2_ref,
                w3_ref, b3_ref, y_ref):
    dn = (((1,), (0,)), ((), ()))
    acc = jnp.concatenate([p_ref[0] + xs_ref[0], p_ref[1] + xs_ref[1]],
                          axis=-1)
    g = lax.dot_general(acc, cw_ref[...], dn,
                        preferred_element_type=jnp.float32)
    g = g * d_ref[...] + cb_ref[...]
    h = jnp.maximum(g, 0.0) + x_ref[...]
    h1 = lax.dot_general(h, w1_ref[...], dn,
                         preferred_element_type=jnp.float32) + b1_ref[...]
    h1 = jnp.maximum(h1, 0.0)
    h2 = lax.dot_general(h1, w2_ref[...], dn,
                         preferred_element_type=jnp.float32) + b2_ref[...]
    h2 = jnp.maximum(h2, 0.0)
    y_ref[...] = lax.dot_general(h2, w3_ref[...], dn,
                                 preferred_element_type=jnp.float32) + b3_ref[...]


def kernel(x, edge_index, conv_W, conv_b, lin1_W, lin1_b, lin2_W, lin2_b,
           lin3_W, lin3_b):
    n, dfeat = x.shape
    e = edge_index.shape[1]
    m = lin1_W.shape[1]
    ept = e // NS
    nchunk = -(-ept // CH)
    nchunk = -(-nchunk // (2 * SLAB)) * (2 * SLAB)
    ept_pad = nchunk * CH
    np_pad = -(-n // (NS * CH)) * (NS * CH)
    if np_pad == n:
        np_pad += NS * CH
    assert ept * NS == e and dfeat == 2 * DH

    src = edge_index[0].reshape(NS, ept)
    dst = edge_index[1].reshape(NS, ept)
    pad_n = ept_pad - ept
    src_g = jnp.concatenate(
        [src, jnp.zeros((NS, pad_n), jnp.int32)], axis=1
    ).reshape(NS, nchunk, CH)
    src_p = jnp.stack([src_g, src_g + np_pad])
    dst_p = jnp.concatenate(
        [dst, jnp.full((NS, pad_n), n, jnp.int32)], axis=1
    ).reshape(NS, nchunk, CH)
    mesh = plsc.VectorSubcoreMesh(core_axis_name="c", subcore_axis_name="s")
    sc_params = pltpu.CompilerParams(needs_layout_passes=False)

    hist = pl.kernel(
        functools.partial(_hist_kernel_body, nchunk=nchunk, np_pad=np_pad),
        out_type=jax.ShapeDtypeStruct((NT, np_pad), jnp.float32),
        mesh=mesh,
        compiler_params=sc_params,
        scratch_types=[
            pltpu.VMEM((nchunk // 2, CH), jnp.int32),
            pltpu.VMEM((np_pad,), jnp.float32),
        ],
    )(dst_p)

    bn = 2560
    grid_b = np_pad // bn
    d_col, xs2 = pl.pallas_call(
        _scale_body,
        grid=(grid_b,),
        in_specs=[
            pl.BlockSpec((NT, bn), lambda i: (0, i)),
            pl.BlockSpec((bn, dfeat), lambda i: (i, 0)),
        ],
        out_specs=[
            pl.BlockSpec((bn, 1), lambda i: (i, 0)),
            pl.BlockSpec((2, bn, DH), lambda i: (0, i, 0)),
        ],
        out_shape=[
            jax.ShapeDtypeStruct((np_pad, 1), jnp.float32),
            jax.ShapeDtypeStruct((2, np_pad, DH), jnp.float32),
        ],
    )(hist, x)
    xs_tab = xs2.reshape(2 * np_pad, DH)

    parts = pl.kernel(
        functools.partial(_scatter_kernel_body, nchunk=nchunk,
                          np_pad=np_pad),
        out_type=jax.ShapeDtypeStruct((NC, np_pad, DH), jnp.float32),
        mesh=mesh,
        compiler_params=pltpu.CompilerParams(needs_layout_passes=False,
                                             use_tc_tiling_on_sc=False),
        scratch_types=[
            pltpu.VMEM((SLAB, CH), jnp.int32),
            pltpu.VMEM((SLAB, CH), jnp.int32),
            pltpu.VMEM((CH, DH), jnp.float32),
            pltpu.VMEM((CH, DH), jnp.float32),
            pltpu.VMEM((CH, DH), jnp.float32),
            pltpu.VMEM((CH, DH), jnp.float32),
            pltpu.VMEM_SHARED((np_pad, DH), jnp.float32),
            pltpu.SemaphoreType.DMA((4,)),
            pltpu.SemaphoreType.DMA((4,)),
        ],
    )(src_p, dst_p, xs_tab)

    bd = 2000
    grid_d = n // bd
    y = pl.pallas_call(
        _dense_body,
        grid=(grid_d,),
        in_specs=[
            pl.BlockSpec((NC, bd, DH), lambda i: (0, i, 0)),
            pl.BlockSpec((2, bd, DH), lambda i: (0, i, 0)),
            pl.BlockSpec((bd, dfeat), lambda i: (i, 0)),
            pl.BlockSpec((bd, 1), lambda i: (i, 0)),
            pl.BlockSpec((dfeat, dfeat), lambda i: (0, 0)),
            pl.BlockSpec((1, dfeat), lambda i: (0, 0)),
            pl.BlockSpec((dfeat, m), lambda i: (0, 0)),
            pl.BlockSpec((1, m), lambda i: (0, 0)),
            pl.BlockSpec((m, m), lambda i: (0, 0)),
            pl.BlockSpec((1, m), lambda i: (0, 0)),
            pl.BlockSpec((m, 1), lambda i: (0, 0)),
            pl.BlockSpec((1, 1), lambda i: (0, 0)),
        ],
        out_specs=pl.BlockSpec((bd, 1), lambda i: (i, 0)),
        out_shape=jax.ShapeDtypeStruct((n, 1), jnp.float32),
    )(parts, xs2, x, d_col,
      conv_W, conv_b.reshape(1, dfeat), lin1_W, lin1_b.reshape(1, m),
      lin2_W, lin2_b.reshape(1, m), lin3_W, lin3_b.reshape(1, 1))
    return y
